# baseline (speedup 1.0000x reference)
import functools
import math

import jax
import jax.numpy as jnp
from jax.experimental import pallas as pl
from jax.experimental.pallas import tpu as pltpu


def _se_dense_body(x_ref, sel_ref, selt_ref, w1t_ref, w2t_ref, o_ref,
                   pooled_s, gseg_s, *, bt, rows, cpr, inv_hw):
    x = x_ref[...]

    seg = jnp.dot(x, sel_ref[...], preferred_element_type=jnp.float32)
    seg3 = seg.reshape(bt, rows, cpr)
    for r in range(rows):
        pooled_s[:, r * cpr:(r + 1) * cpr] = seg3[:, r, :]
    pooled = pooled_s[...] * inv_hw

    hid = jnp.dot(pooled, w1t_ref[...], preferred_element_type=jnp.float32)
    hid = jnp.maximum(hid, 0.0)
    gate = jax.nn.sigmoid(
        jnp.dot(hid, w2t_ref[...], preferred_element_type=jnp.float32))

    for r in range(rows):
        gseg_s[:, r, :] = gate[:, r * cpr:(r + 1) * cpr]
    grk = gseg_s[...].reshape(bt * rows, cpr)
    glanes = jnp.dot(grk, selt_ref[...], preferred_element_type=jnp.float32)
    o_ref[...] = x * glanes.astype(x.dtype)


def _se_strided_body(x_ref, w1t_ref, w2t_ref, o_ref, *, inv_hw):
    x = x_ref[...]
    pooled = jnp.sum(x, axis=2, dtype=jnp.float32) * inv_hw
    hid = jnp.maximum(
        jnp.dot(pooled, w1t_ref[...], preferred_element_type=jnp.float32), 0.0)
    gate = jax.nn.sigmoid(
        jnp.dot(hid, w2t_ref[...], preferred_element_type=jnp.float32))
    o_ref[...] = x * gate.astype(x.dtype)[:, :, None]


def _pick_bt(B, slab_bytes, budget_bytes):
    fit = max(int(budget_bytes // (4 * slab_bytes)), 1)
    bt = min(fit, max(B // 4, 1))
    while bt > 1 and B % bt:
        bt -= 1
    return bt


def kernel(x_nchw, w1, w2):
    B, C, H, W = x_nchw.shape
    hidden = w1.shape[0]
    HW = H * W
    dt = x_nchw.dtype
    itemsize = jnp.dtype(dt).itemsize

    w1t = w1.T.astype(jnp.float32)
    w2t = w2.T.astype(jnp.float32)

    R = math.lcm(HW, 128)
    cpr = R // HW
    dense_ok = (C % cpr == 0) and ((C // cpr) % 8 == 0) and R * itemsize <= (1 << 20)

    cost = pl.CostEstimate(
        flops=B * (4 * C * HW + 4 * C * hidden),
        transcendentals=B * C,
        bytes_accessed=2 * B * C * HW * itemsize,
    )

    if dense_ok:
        rows = C // cpr
        x_v = x_nchw.reshape(B * rows, R)

        lane = jax.lax.broadcasted_iota(jnp.int32, (R, cpr), 0) // HW
        slot = jax.lax.broadcasted_iota(jnp.int32, (R, cpr), 1)
        sel = (lane == slot).astype(jnp.float32)
        selt = sel.T

        bt = _pick_bt(B, rows * R * itemsize, 40 << 20)
        body = functools.partial(_se_dense_body, bt=bt, rows=rows, cpr=cpr,
                                 inv_hw=1.0 / HW)
        out_v = pl.pallas_call(
            body,
            out_shape=jax.ShapeDtypeStruct((B * rows, R), dt),
            grid=(B // bt,),
            in_specs=[
                pl.BlockSpec((bt * rows, R), lambda b: (b, 0)),
                pl.BlockSpec((R, cpr), lambda b: (0, 0)),
                pl.BlockSpec((cpr, R), lambda b: (0, 0)),
                pl.BlockSpec((C, hidden), lambda b: (0, 0)),
                pl.BlockSpec((hidden, C), lambda b: (0, 0)),
            ],
            out_specs=pl.BlockSpec((bt * rows, R), lambda b: (b, 0)),
            scratch_shapes=[
                pltpu.VMEM((bt, C), jnp.float32),
                pltpu.VMEM((bt, rows, cpr), jnp.float32),
            ],
            compiler_params=pltpu.CompilerParams(
                dimension_semantics=("parallel",),
                vmem_limit_bytes=60 << 20,
            ),
            cost_estimate=cost,
        )(x_v, sel, selt, w1t, w2t)
        return out_v.reshape(B, C, H, W)

    x_flat = x_nchw.reshape(B, C, HW)
    lanes_pad = -(-HW // 128) * 128
    bt = _pick_bt(B, C * lanes_pad * itemsize, 40 << 20)
    out_flat = pl.pallas_call(
        functools.partial(_se_strided_body, inv_hw=1.0 / HW),
        out_shape=jax.ShapeDtypeStruct((B, C, HW), dt),
        grid=(B // bt,),
        in_specs=[
            pl.BlockSpec((bt, C, HW), lambda b: (b, 0, 0)),
            pl.BlockSpec((C, hidden), lambda b: (0, 0)),
            pl.BlockSpec((hidden, C), lambda b: (0, 0)),
        ],
        out_specs=pl.BlockSpec((bt, C, HW), lambda b: (b, 0, 0)),
        compiler_params=pltpu.CompilerParams(
            dimension_semantics=("parallel",),
            vmem_limit_bytes=60 << 20,
        ),
        cost_estimate=cost,
    )(x_flat, w1t, w2t)
    return out_flat.reshape(B, C, H, W)


# speedup vs baseline: 3.5993x; 3.5993x over previous
import functools

import jax
import jax.numpy as jnp
from jax.experimental import pallas as pl
from jax.experimental.pallas import tpu as pltpu

_DEPTH = 4


def _se_ring_body(x_hbm, w1t_ref, w2t_ref, o_hbm, x_buf, o_buf,
                  in_sems, out_sems, *, bt, steps, inv_hw):
    base = pl.program_id(0) * steps

    def dma_in(slot, step):
        pltpu.make_async_copy(x_hbm.at[pl.ds((base + step) * bt, bt)],
                              x_buf.at[slot], in_sems.at[slot]).start()

    def wait_in(slot):
        pltpu.make_async_copy(x_buf.at[slot], x_buf.at[slot],
                              in_sems.at[slot]).wait()

    def dma_out(slot, step):
        pltpu.make_async_copy(o_buf.at[slot],
                              o_hbm.at[pl.ds((base + step) * bt, bt)],
                              out_sems.at[slot]).start()

    def wait_out(slot):
        pltpu.make_async_copy(o_buf.at[slot], o_buf.at[slot],
                              out_sems.at[slot]).wait()

    for k in range(min(_DEPTH, steps)):
        dma_in(k, k)

    for i in range(steps):
        slot = i % _DEPTH
        wait_in(slot)
        if i >= _DEPTH:
            wait_out(slot)

        x = x_buf[slot]
        pooled = jnp.sum(x, axis=2, dtype=jnp.float32) * inv_hw
        hid = jnp.maximum(
            jnp.dot(pooled, w1t_ref[...],
                    preferred_element_type=jnp.float32), 0.0)
        gate = jax.nn.sigmoid(
            jnp.dot(hid, w2t_ref[...], preferred_element_type=jnp.float32))
        o_buf[slot] = x * gate.astype(x.dtype)[:, :, None]

        dma_out(slot, i)
        if i + _DEPTH < steps:
            dma_in(slot, i + _DEPTH)

    for i in range(max(steps - _DEPTH, 0), steps):
        wait_out(i % _DEPTH)


def kernel(x_nchw, w1, w2):
    B, C, H, W = x_nchw.shape
    hidden = w1.shape[0]
    HW = H * W
    dt = x_nchw.dtype

    x_flat = x_nchw.reshape(B, C, HW)
    w1t = w1.T.astype(jnp.float32)
    w2t = w2.T.astype(jnp.float32)

    ncores = 2 if B % 2 == 0 else 1
    bt = 2 if B % (2 * ncores) == 0 else 1
    steps = B // (bt * ncores)

    cost = pl.CostEstimate(
        flops=B * (4 * C * HW + 4 * C * hidden),
        transcendentals=B * C,
        bytes_accessed=2 * B * C * HW * jnp.dtype(dt).itemsize,
    )

    out_flat = pl.pallas_call(
        functools.partial(_se_ring_body, bt=bt, steps=steps, inv_hw=1.0 / HW),
        out_shape=jax.ShapeDtypeStruct((B, C, HW), dt),
        grid=(ncores,),
        in_specs=[
            pl.BlockSpec(memory_space=pl.ANY),
            pl.BlockSpec((C, hidden), lambda p: (0, 0)),
            pl.BlockSpec((hidden, C), lambda p: (0, 0)),
        ],
        out_specs=pl.BlockSpec(memory_space=pl.ANY),
        scratch_shapes=[
            pltpu.VMEM((_DEPTH, bt, C, HW), dt),
            pltpu.VMEM((_DEPTH, bt, C, HW), dt),
            pltpu.SemaphoreType.DMA((_DEPTH,)),
            pltpu.SemaphoreType.DMA((_DEPTH,)),
        ],
        compiler_params=pltpu.CompilerParams(
            dimension_semantics=("parallel",),
            vmem_limit_bytes=60 << 20,
        ),
        cost_estimate=cost,
    )(x_flat, w1t, w2t)

    return out_flat.reshape(B, C, H, W)


# speedup vs baseline: 3.6112x; 1.0033x over previous
import functools

import jax
import jax.numpy as jnp
from jax.experimental import pallas as pl
from jax.experimental.pallas import tpu as pltpu

_DEPTH = 4


def _se_ring_body(x_hbm, w1t_ref, w2t_ref, o_hbm, x_buf, o_buf,
                  in_sems, out_sems, *, bt, steps, inv_hw):
    base = pl.program_id(0) * steps

    def dma_in(slot, step):
        pltpu.make_async_copy(x_hbm.at[pl.ds((base + step) * bt, bt)],
                              x_buf.at[slot], in_sems.at[slot]).start()

    def wait_in(slot):
        pltpu.make_async_copy(x_buf.at[slot], x_buf.at[slot],
                              in_sems.at[slot]).wait()

    def dma_out(slot, step):
        pltpu.make_async_copy(o_buf.at[slot],
                              o_hbm.at[pl.ds((base + step) * bt, bt)],
                              out_sems.at[slot]).start(priority=1)

    def wait_out(slot):
        pltpu.make_async_copy(o_buf.at[slot], o_buf.at[slot],
                              out_sems.at[slot]).wait()

    for k in range(min(_DEPTH, steps)):
        dma_in(k, k)

    for i in range(steps):
        slot = i % _DEPTH
        wait_in(slot)
        if i >= _DEPTH:
            wait_out(slot)

        x = x_buf[slot]
        pooled = jnp.sum(x, axis=2, dtype=jnp.float32) * inv_hw
        hid = jnp.maximum(
            jnp.dot(pooled, w1t_ref[...],
                    preferred_element_type=jnp.float32), 0.0)
        gate = jax.nn.sigmoid(
            jnp.dot(hid, w2t_ref[...], preferred_element_type=jnp.float32))
        o_buf[slot] = x * gate.astype(x.dtype)[:, :, None]

        dma_out(slot, i)
        if i + _DEPTH < steps:
            dma_in(slot, i + _DEPTH)

    for i in range(max(steps - _DEPTH, 0), steps):
        wait_out(i % _DEPTH)


def kernel(x_nchw, w1, w2):
    B, C, H, W = x_nchw.shape
    hidden = w1.shape[0]
    HW = H * W
    dt = x_nchw.dtype

    x_flat = x_nchw.reshape(B, C, HW)
    w1t = w1.T.astype(jnp.float32)
    w2t = w2.T.astype(jnp.float32)

    ncores = 2 if B % 2 == 0 else 1
    bt = 2 if B % (2 * ncores) == 0 else 1
    steps = B // (bt * ncores)

    cost = pl.CostEstimate(
        flops=B * (4 * C * HW + 4 * C * hidden),
        transcendentals=B * C,
        bytes_accessed=2 * B * C * HW * jnp.dtype(dt).itemsize,
    )

    out_flat = pl.pallas_call(
        functools.partial(_se_ring_body, bt=bt, steps=steps, inv_hw=1.0 / HW),
        out_shape=jax.ShapeDtypeStruct((B, C, HW), dt),
        grid=(ncores,),
        in_specs=[
            pl.BlockSpec(memory_space=pl.ANY),
            pl.BlockSpec((C, hidden), lambda p: (0, 0)),
            pl.BlockSpec((hidden, C), lambda p: (0, 0)),
        ],
        out_specs=pl.BlockSpec(memory_space=pl.ANY),
        scratch_shapes=[
            pltpu.VMEM((_DEPTH, bt, C, HW), dt),
            pltpu.VMEM((_DEPTH, bt, C, HW), dt),
            pltpu.SemaphoreType.DMA((_DEPTH,)),
            pltpu.SemaphoreType.DMA((_DEPTH,)),
        ],
        compiler_params=pltpu.CompilerParams(
            dimension_semantics=("parallel",),
            vmem_limit_bytes=60 << 20,
        ),
        cost_estimate=cost,
    )(x_flat, w1t, w2t)

    return out_flat.reshape(B, C, H, W)
